# Initial kernel scaffold; baseline (speedup 1.0000x reference)
import jax
import jax.numpy as jnp
from jax.experimental import pallas as pl


def kernel(nodes, edges, graph_globals, W1, b1, W2, b2, edge_index, batch_edges):
    raise NotImplementedError("write your pallas kernel here")


# speedup vs baseline: 2.1752x; 2.1752x over previous
import functools

import jax
import jax.numpy as jnp
from jax import lax
from jax.experimental import pallas as pl
from jax.experimental.pallas import tpu as pltpu
from jax.experimental.pallas import tpu_sc as plsc

F32 = jnp.float32
_PREC = lax.Precision.HIGHEST

_N_NODES = 10000
_N_EDGES = 320000
_EDGE_DIM = 16
_NODE_DIM = 128
_HID = 64
_N_GRAPHS = 16

_NC = 2
_NS = 16
_NW = _NC * _NS
_EPW = _N_EDGES // _NW
_CHUNK = 80
_NCHUNK = _EPW // _CHUNK

_BE = 2560
_NBLK = _N_EDGES // _BE


def _prep_body(nodes_ref, w1s_ref, w1d_ref, gp_ref, w1u_ref, b1_ref,
               ps_ref, pd_ref, g_ref):
    n = nodes_ref[...]
    ps_ref[...] = jnp.dot(n, w1s_ref[...], precision=_PREC,
                          preferred_element_type=F32)
    pd_ref[...] = jnp.dot(n, w1d_ref[...], precision=_PREC,
                          preferred_element_type=F32)
    g_ref[...] = jnp.dot(gp_ref[...], w1u_ref[...], precision=_PREC,
                         preferred_element_type=F32) + b1_ref[...]


def _gather_body(ps_hbm, pd_hbm, send_hbm, recv_hbm, out_hbm,
                 idx0, idx1, rows0, rows1, sem0, sem1):
    wid = lax.axis_index("s") * _NC + lax.axis_index("c")
    base = wid * _EPW

    @pl.loop(0, _NCHUNK)
    def _chunk(c):
        off = base + c * _CHUNK
        pltpu.sync_copy(send_hbm.at[pl.ds(off, _CHUNK)], idx0)
        pltpu.sync_copy(recv_hbm.at[pl.ds(off, _CHUNK)], idx1)
        cp0 = pltpu.async_copy(ps_hbm.at[idx0], rows0, sem0)
        cp1 = pltpu.async_copy(pd_hbm.at[idx1], rows1, sem1)
        cp0.wait()
        cp1.wait()

        @pl.loop(0, _CHUNK)
        def _row(i):
            for j in range(_HID // 16):
                sl = pl.ds(j * 16, 16)
                plsc.addupdate(rows0.at[i, sl], rows1[i, sl])

        pltpu.sync_copy(rows0, out_hbm.at[pl.ds(off, _CHUNK)])


def _mlp_body(s_ref, e_ref, b_ref, g_ref, w1e_ref, w2_ref, b2_ref, o_ref):
    onehot = (b_ref[...] == lax.broadcasted_iota(jnp.int32, (_BE, 128), 1)
              ).astype(F32)
    x = (s_ref[...]
         + jnp.dot(e_ref[...], w1e_ref[...], precision=_PREC,
                   preferred_element_type=F32)
         + jnp.dot(onehot, g_ref[...], precision=_PREC,
                   preferred_element_type=F32))
    h = jnp.maximum(x, 0.0)
    o_ref[...] = jnp.dot(h, w2_ref[...], precision=_PREC,
                         preferred_element_type=F32) + b2_ref[...]


def kernel(nodes, edges, graph_globals, W1, b1, W2, b2, edge_index,
           batch_edges):
    W1e = W1[:_EDGE_DIM]
    W1s = W1[_EDGE_DIM:_EDGE_DIM + _NODE_DIM]
    W1d = W1[_EDGE_DIM + _NODE_DIM:_EDGE_DIM + 2 * _NODE_DIM]
    W1u = W1[_EDGE_DIM + 2 * _NODE_DIM:]
    gp = jnp.zeros((128, W1u.shape[0]), F32).at[:_N_GRAPHS].set(graph_globals)
    send = edge_index[0]
    recv = edge_index[1]
    batch2d = batch_edges.reshape(_N_EDGES, 1)
    b1r = b1.reshape(1, _HID)
    b2r = b2.reshape(1, _EDGE_DIM)

    ps, pd, g = pl.pallas_call(
        _prep_body,
        out_shape=[
            jax.ShapeDtypeStruct((_N_NODES, _HID), F32),
            jax.ShapeDtypeStruct((_N_NODES, _HID), F32),
            jax.ShapeDtypeStruct((128, _HID), F32),
        ],
    )(nodes, W1s, W1d, gp, W1u, b1r)

    mesh = plsc.VectorSubcoreMesh(core_axis_name="c", subcore_axis_name="s",
                                  num_cores=_NC, num_subcores=_NS)
    gather = functools.partial(
        pl.kernel,
        mesh=mesh,
        compiler_params=pltpu.CompilerParams(use_tc_tiling_on_sc=False),
        out_type=jax.ShapeDtypeStruct((_N_EDGES, _HID), F32),
        scratch_types=[
            pltpu.VMEM((_CHUNK,), jnp.int32),
            pltpu.VMEM((_CHUNK,), jnp.int32),
            pltpu.VMEM((_CHUNK, _HID), F32),
            pltpu.VMEM((_CHUNK, _HID), F32),
            pltpu.SemaphoreType.DMA,
            pltpu.SemaphoreType.DMA,
        ],
    )(_gather_body)
    s = gather(ps, pd, send, recv)

    out = pl.pallas_call(
        _mlp_body,
        grid=(_NBLK,),
        in_specs=[
            pl.BlockSpec((_BE, _HID), lambda i: (i, 0)),
            pl.BlockSpec((_BE, _EDGE_DIM), lambda i: (i, 0)),
            pl.BlockSpec((_BE, 1), lambda i: (i, 0)),
            pl.BlockSpec((128, _HID), lambda i: (0, 0)),
            pl.BlockSpec((_EDGE_DIM, _HID), lambda i: (0, 0)),
            pl.BlockSpec((_HID, _EDGE_DIM), lambda i: (0, 0)),
            pl.BlockSpec((1, _EDGE_DIM), lambda i: (0, 0)),
        ],
        out_specs=pl.BlockSpec((_BE, _EDGE_DIM), lambda i: (i, 0)),
        out_shape=jax.ShapeDtypeStruct((_N_EDGES, _EDGE_DIM), F32),
    )(s, edges, batch2d, g, W1e, W2, b2r)
    return out


# speedup vs baseline: 3.2898x; 1.5124x over previous
import functools

import jax
import jax.numpy as jnp
from jax import lax
from jax.experimental import pallas as pl
from jax.experimental.pallas import tpu as pltpu
from jax.experimental.pallas import tpu_sc as plsc

F32 = jnp.float32
_PREC = lax.Precision.DEFAULT

_N_NODES = 10000
_N_EDGES = 320000
_EDGE_DIM = 16
_NODE_DIM = 128
_HID = 64
_N_GRAPHS = 16

_NC = 2
_NS = 16
_NW = _NC * _NS
_EPW = _N_EDGES // _NW
_CHUNK = 80
_NCHUNK = _EPW // _CHUNK

_BE = 2560
_NBLK = _N_EDGES // _BE


def _prep_body(nodes_ref, w1s_ref, w1d_ref, gp_ref, w1u_ref, b1_ref,
               ps_ref, pd_ref, g_ref):
    n = nodes_ref[...]
    ps_ref[...] = jnp.dot(n, w1s_ref[...], precision=_PREC,
                          preferred_element_type=F32)
    pd_ref[...] = jnp.dot(n, w1d_ref[...], precision=_PREC,
                          preferred_element_type=F32)
    g_ref[...] = jnp.dot(gp_ref[...], w1u_ref[...], precision=_PREC,
                         preferred_element_type=F32) + b1_ref[...]


def _gather_body(ps_hbm, pd_hbm, send_hbm, recv_hbm, out_hbm,
                 idx0, idx1, rows0, rows1, sem0, sem1):
    wid = lax.axis_index("s") * _NC + lax.axis_index("c")
    base = wid * _EPW

    @pl.loop(0, _NCHUNK)
    def _chunk(c):
        off = base + c * _CHUNK
        pltpu.sync_copy(send_hbm.at[pl.ds(off, _CHUNK)], idx0)
        pltpu.sync_copy(recv_hbm.at[pl.ds(off, _CHUNK)], idx1)
        cp0 = pltpu.async_copy(ps_hbm.at[idx0], rows0, sem0)
        cp1 = pltpu.async_copy(pd_hbm.at[idx1], rows1, sem1)
        cp0.wait()
        cp1.wait()

        @pl.loop(0, _CHUNK)
        def _row(i):
            for j in range(_HID // 16):
                sl = pl.ds(j * 16, 16)
                plsc.addupdate(rows0.at[i, sl], rows1[i, sl])

        pltpu.sync_copy(rows0, out_hbm.at[pl.ds(off, _CHUNK)])


def _mlp_body(s_ref, e_ref, b_ref, g_ref, w1e_ref, w2_ref, b2_ref, o_ref):
    onehot = (b_ref[...] == lax.broadcasted_iota(jnp.int32, (_BE, _N_GRAPHS), 1)
              ).astype(F32)
    x = (s_ref[...]
         + jnp.dot(e_ref[...], w1e_ref[...], precision=_PREC,
                   preferred_element_type=F32)
         + jnp.dot(onehot, g_ref[...], precision=_PREC,
                   preferred_element_type=F32))
    h = jnp.maximum(x, 0.0)
    o_ref[...] = jnp.dot(h, w2_ref[...], precision=_PREC,
                         preferred_element_type=F32) + b2_ref[...]


def kernel(nodes, edges, graph_globals, W1, b1, W2, b2, edge_index,
           batch_edges):
    W1e = W1[:_EDGE_DIM]
    W1s = W1[_EDGE_DIM:_EDGE_DIM + _NODE_DIM]
    W1d = W1[_EDGE_DIM + _NODE_DIM:_EDGE_DIM + 2 * _NODE_DIM]
    W1u = W1[_EDGE_DIM + 2 * _NODE_DIM:]
    gp = graph_globals
    send = edge_index[0]
    recv = edge_index[1]
    batch2d = batch_edges.reshape(_N_EDGES, 1)
    b1r = b1.reshape(1, _HID)
    b2r = b2.reshape(1, _EDGE_DIM)

    ps, pd, g = pl.pallas_call(
        _prep_body,
        out_shape=[
            jax.ShapeDtypeStruct((_N_NODES, _HID), F32),
            jax.ShapeDtypeStruct((_N_NODES, _HID), F32),
            jax.ShapeDtypeStruct((_N_GRAPHS, _HID), F32),
        ],
    )(nodes, W1s, W1d, gp, W1u, b1r)

    mesh = plsc.VectorSubcoreMesh(core_axis_name="c", subcore_axis_name="s",
                                  num_cores=_NC, num_subcores=_NS)
    gather = functools.partial(
        pl.kernel,
        mesh=mesh,
        compiler_params=pltpu.CompilerParams(use_tc_tiling_on_sc=False),
        out_type=jax.ShapeDtypeStruct((_N_EDGES, _HID), F32),
        scratch_types=[
            pltpu.VMEM((_CHUNK,), jnp.int32),
            pltpu.VMEM((_CHUNK,), jnp.int32),
            pltpu.VMEM((_CHUNK, _HID), F32),
            pltpu.VMEM((_CHUNK, _HID), F32),
            pltpu.SemaphoreType.DMA,
            pltpu.SemaphoreType.DMA,
        ],
    )(_gather_body)
    s = gather(ps, pd, send, recv)

    out = pl.pallas_call(
        _mlp_body,
        grid=(_NBLK,),
        in_specs=[
            pl.BlockSpec((_BE, _HID), lambda i: (i, 0)),
            pl.BlockSpec((_BE, _EDGE_DIM), lambda i: (i, 0)),
            pl.BlockSpec((_BE, 1), lambda i: (i, 0)),
            pl.BlockSpec((_N_GRAPHS, _HID), lambda i: (0, 0)),
            pl.BlockSpec((_EDGE_DIM, _HID), lambda i: (0, 0)),
            pl.BlockSpec((_HID, _EDGE_DIM), lambda i: (0, 0)),
            pl.BlockSpec((1, _EDGE_DIM), lambda i: (0, 0)),
        ],
        out_specs=pl.BlockSpec((_BE, _EDGE_DIM), lambda i: (i, 0)),
        out_shape=jax.ShapeDtypeStruct((_N_EDGES, _EDGE_DIM), F32),
    )(s, edges, batch2d, g, W1e, W2, b2r)
    return out
